# baseline (speedup 1.0000x reference)
import jax
import jax.numpy as jnp
from jax.experimental import pallas as pl
from jax.experimental.pallas import tpu as pltpu

N = 512
STEPS = 12
IB = 128
F32 = jnp.float32


def _dot(a, b):
    return jax.lax.dot_general(a, b, (((1,), (0,)), ((), ())),
                               preferred_element_type=F32)


def _decoder_kernel(nei_ref, eps_ref, c_ref, z_ref, p0_ref, lp_ref, h0_ref,
                    cc0_ref, Wp_ref, Wc_ref, Wz_ref, Wx_ref, bin_ref,
                    Wih_ref, Whh_ref, bg_ref, M2_ref, WI_ref, WJ_ref,
                    cst_ref, Wm2_ref, bm2_ref, Wl1_ref, Wl2_ref, blat_ref,
                    out_ref, h_s, cc_s, ctx_s, p_s, prev_s, A_s):
    t = pl.program_id(0)

    @pl.when(t == 0)
    def _init():
        h_s[...] = h0_ref[...]
        cc_s[...] = cc0_ref[...]
        ctx_s[...] = jnp.zeros((N, 8), F32)
        p_s[...] = p0_ref[...]
        prev_s[...] = lp_ref[...]

    h = h_s[...]
    cc = cc_s[...]
    ctx = ctx_s[...]
    p = p_s[...]
    prev = prev_s[...]

    emb = jax.nn.relu(_dot(prev, Wp_ref[...]) + _dot(c_ref[...], Wc_ref[...])
                      + _dot(z_ref[...], Wz_ref[...]) + _dot(ctx, Wx_ref[...])
                      + bin_ref[...])

    gates = _dot(emb, Wih_ref[...]) + _dot(h, Whh_ref[...]) + bg_ref[...]
    ig = jax.nn.sigmoid(gates[:, 0:8])
    fg = jax.nn.sigmoid(gates[:, 8:16])
    gg = jnp.tanh(gates[:, 16:24])
    og = jax.nn.sigmoid(gates[:, 24:32])
    cc = fg * cc + ig * gg
    h = og * jnp.tanh(cc)

    pm = _dot(p, M2_ref[...])
    A_s[...] = pm + _dot(h, WI_ref[...]) + cst_ref[...]
    B = _dot(h, WJ_ref[...]) - pm
    Bt = jnp.transpose(jnp.concatenate([B, B], axis=1))

    W2T = Wm2_ref[...]
    bm2 = bm2_ref[...]
    HN = N // 2

    def _block(b, carry):
        lo = b * IB
        Ab = jnp.concatenate([A_s[pl.ds(lo, IB), :],
                              A_s[pl.ds(HN + lo, IB), :]], axis=1)
        AbT = jnp.transpose(Ab)
        mk_lo = nei_ref[0, pl.ds(lo, IB), :]
        mk_hi = nei_ref[0, pl.ds(HN + lo, IB), :]
        x1big = jnp.concatenate(
            [jax.nn.relu(AbT[:, i2:i2 + 1] + Bt).astype(jnp.bfloat16)
             for i2 in range(IB)],
            axis=1)
        x2big = _dot(W2T, x1big)
        pools = []
        for i2 in range(IB):
            x2t = x2big[:, i2 * N:(i2 + 1) * N]
            m16 = jnp.concatenate(
                [jnp.broadcast_to(mk_lo[i2][None, :], (8, N)),
                 jnp.broadcast_to(mk_hi[i2][None, :], (8, N))], axis=0)
            x2m = jnp.where(m16 > 0, x2t, -1e9)
            pools.append(jnp.max(x2m, axis=1))
        pool = jax.nn.relu(jnp.stack(pools, axis=1) + bm2)
        ctx_s[pl.ds(lo, IB), :] = jnp.transpose(pool[0:8, :])
        ctx_s[pl.ds(HN + lo, IB), :] = jnp.transpose(pool[8:16, :])
        return carry

    jax.lax.fori_loop(0, HN // IB, _block, 0)

    ctx = ctx_s[...]

    mean = h[:, 0:4]
    logvar = h[:, 4:8]
    lat = mean + eps_ref[0] * jnp.exp(0.5 * logvar)
    out = _dot(lat, Wl1_ref[...]) + _dot(ctx, Wl2_ref[...]) + blat_ref[...]

    out_ref[0] = out
    p_s[...] = p + out
    prev_s[...] = out
    h_s[...] = h
    cc_s[...] = cc


def kernel(last_position, c, z, obs_traj_pos, nei_index, nei_num_index,
           W_in, b_in, W_ih, W_hh, b_ih, b_hh, W_lat, b_lat, W_sp, b_sp,
           W_m1, b_m1, W_m2, b_m2):
    k1, k2, k3 = jax.random.split(jax.random.key(1), 3)
    h0 = jax.random.normal(k1, (N, 8), dtype=F32)
    cc0 = jax.random.normal(k2, (N, 8), dtype=F32)
    eps = jnp.stack([jax.random.normal(jax.random.fold_in(k3, i), (N, 4),
                                       dtype=F32) for i in range(STEPS)])
    p0 = obs_traj_pos[-1]

    Wp = W_in[0:2]
    Wc = W_in[2:18]
    Wz = W_in[18:34]
    Wx = W_in[34:42]
    bin_ = b_in.reshape(1, 32)
    bg = (b_ih + b_hh).reshape(1, 32)
    M2 = W_sp @ W_m1[0:32]
    cst = (b_sp @ W_m1[0:32] + b_m1).reshape(1, 64)
    WJ = W_m1[32:40]
    WI = W_m1[40:48]
    W2blk = jnp.zeros((128, 16), F32)
    W2blk = W2blk.at[0:64, 0:8].set(W_m2).at[64:128, 8:16].set(W_m2)
    W2blk = W2blk.T.astype(jnp.bfloat16)
    bm2 = jnp.concatenate([b_m2, b_m2]).reshape(16, 1)
    Wl1 = W_lat[0:4]
    Wl2 = W_lat[4:12]
    blat = b_lat.reshape(1, 2)

    def full(x):
        nd = x.ndim
        return pl.BlockSpec(x.shape, lambda t, _nd=nd: (0,) * _nd)

    operands = (nei_index, eps, c, z, p0, last_position, h0, cc0,
                Wp, Wc, Wz, Wx, bin_, W_ih, W_hh, bg, M2, WI, WJ, cst,
                W2blk, bm2, Wl1, Wl2, blat)
    in_specs = [
        pl.BlockSpec((1, N, N), lambda t: (t, 0, 0)),
        pl.BlockSpec((1, N, 4), lambda t: (t, 0, 0)),
    ] + [full(x) for x in operands[2:]]

    out = pl.pallas_call(
        _decoder_kernel,
        grid=(STEPS,),
        in_specs=in_specs,
        out_specs=pl.BlockSpec((1, N, 2), lambda t: (t, 0, 0)),
        out_shape=jax.ShapeDtypeStruct((STEPS, N, 2), F32),
        scratch_shapes=[
            pltpu.VMEM((N, 8), F32),
            pltpu.VMEM((N, 8), F32),
            pltpu.VMEM((N, 8), F32),
            pltpu.VMEM((N, 2), F32),
            pltpu.VMEM((N, 2), F32),
            pltpu.VMEM((N, 64), F32),
        ],
        compiler_params=pltpu.CompilerParams(
            dimension_semantics=("arbitrary",)),
    )(*operands)
    return out


# speedup vs baseline: 1.0249x; 1.0249x over previous
import jax
import jax.numpy as jnp
from jax.experimental import pallas as pl
from jax.experimental.pallas import tpu as pltpu

N = 512
STEPS = 12
IB = 128
F32 = jnp.float32


def _dot(a, b):
    return jax.lax.dot_general(a, b, (((1,), (0,)), ((), ())),
                               preferred_element_type=F32)


def _decoder_kernel(nei_ref, eps_ref, c_ref, z_ref, p0_ref, lp_ref, h0_ref,
                    cc0_ref, Wp_ref, Wc_ref, Wz_ref, Wx_ref, bin_ref,
                    Wih_ref, Whh_ref, bg_ref, M2_ref, WI_ref, WJ_ref,
                    cst_ref, Wm2_ref, bm2_ref, Wl1_ref, Wl2_ref, blat_ref,
                    out_ref, h_s, cc_s, ctx_s, p_s, prev_s, A_s):
    t = pl.program_id(0)

    @pl.when(t == 0)
    def _init():
        h_s[...] = h0_ref[...]
        cc_s[...] = cc0_ref[...]
        ctx_s[...] = jnp.zeros((N, 8), F32)
        p_s[...] = p0_ref[...]
        prev_s[...] = lp_ref[...]

    h = h_s[...]
    cc = cc_s[...]
    ctx = ctx_s[...]
    p = p_s[...]
    prev = prev_s[...]

    emb = jax.nn.relu(_dot(prev, Wp_ref[...]) + _dot(c_ref[...], Wc_ref[...])
                      + _dot(z_ref[...], Wz_ref[...]) + _dot(ctx, Wx_ref[...])
                      + bin_ref[...])

    gates = _dot(emb, Wih_ref[...]) + _dot(h, Whh_ref[...]) + bg_ref[...]
    ig = jax.nn.sigmoid(gates[:, 0:8])
    fg = jax.nn.sigmoid(gates[:, 8:16])
    gg = jnp.tanh(gates[:, 16:24])
    og = jax.nn.sigmoid(gates[:, 24:32])
    cc = fg * cc + ig * gg
    h = og * jnp.tanh(cc)

    pm = _dot(p, M2_ref[...])
    A_s[...] = pm + _dot(h, WI_ref[...]) + cst_ref[...]
    B = _dot(h, WJ_ref[...]) - pm
    Bt = jnp.transpose(jnp.concatenate([B, B], axis=1))

    W2T = Wm2_ref[...]
    bm2 = bm2_ref[...]
    HN = N // 2

    def _block(b, carry):
        lo = b * IB
        Ab = jnp.concatenate([A_s[pl.ds(lo, IB), :],
                              A_s[pl.ds(HN + lo, IB), :]], axis=1)
        AbT = jnp.transpose(Ab)
        mk_lo = nei_ref[0, pl.ds(lo, IB), :]
        mk_hi = nei_ref[0, pl.ds(HN + lo, IB), :]
        x1big = jnp.concatenate(
            [jax.nn.relu(AbT[:, i2:i2 + 1] + Bt) for i2 in range(IB)],
            axis=1)
        x2big = _dot(W2T, x1big)
        pools = []
        for i2 in range(IB):
            x2t = x2big[:, i2 * N:(i2 + 1) * N]
            m16 = jnp.concatenate(
                [jnp.broadcast_to(mk_lo[i2][None, :], (8, N)),
                 jnp.broadcast_to(mk_hi[i2][None, :], (8, N))], axis=0)
            x2m = jnp.where(m16 > 0, x2t, -1e9)
            pools.append(jnp.max(x2m, axis=1))
        pool = jax.nn.relu(jnp.stack(pools, axis=1) + bm2)
        ctx_s[pl.ds(lo, IB), :] = jnp.transpose(pool[0:8, :])
        ctx_s[pl.ds(HN + lo, IB), :] = jnp.transpose(pool[8:16, :])
        return carry

    jax.lax.fori_loop(0, HN // IB, _block, 0)

    ctx = ctx_s[...]

    mean = h[:, 0:4]
    logvar = h[:, 4:8]
    lat = mean + eps_ref[0] * jnp.exp(0.5 * logvar)
    out = _dot(lat, Wl1_ref[...]) + _dot(ctx, Wl2_ref[...]) + blat_ref[...]

    out_ref[0] = out
    p_s[...] = p + out
    prev_s[...] = out
    h_s[...] = h
    cc_s[...] = cc


def kernel(last_position, c, z, obs_traj_pos, nei_index, nei_num_index,
           W_in, b_in, W_ih, W_hh, b_ih, b_hh, W_lat, b_lat, W_sp, b_sp,
           W_m1, b_m1, W_m2, b_m2):
    k1, k2, k3 = jax.random.split(jax.random.key(1), 3)
    h0 = jax.random.normal(k1, (N, 8), dtype=F32)
    cc0 = jax.random.normal(k2, (N, 8), dtype=F32)
    eps = jnp.stack([jax.random.normal(jax.random.fold_in(k3, i), (N, 4),
                                       dtype=F32) for i in range(STEPS)])
    p0 = obs_traj_pos[-1]

    Wp = W_in[0:2]
    Wc = W_in[2:18]
    Wz = W_in[18:34]
    Wx = W_in[34:42]
    bin_ = b_in.reshape(1, 32)
    bg = (b_ih + b_hh).reshape(1, 32)
    M2 = W_sp @ W_m1[0:32]
    cst = (b_sp @ W_m1[0:32] + b_m1).reshape(1, 64)
    WJ = W_m1[32:40]
    WI = W_m1[40:48]
    W2blk = jnp.zeros((128, 16), F32)
    W2blk = W2blk.at[0:64, 0:8].set(W_m2).at[64:128, 8:16].set(W_m2)
    W2blk = W2blk.T
    bm2 = jnp.concatenate([b_m2, b_m2]).reshape(16, 1)
    Wl1 = W_lat[0:4]
    Wl2 = W_lat[4:12]
    blat = b_lat.reshape(1, 2)

    def full(x):
        nd = x.ndim
        return pl.BlockSpec(x.shape, lambda t, _nd=nd: (0,) * _nd)

    operands = (nei_index, eps, c, z, p0, last_position, h0, cc0,
                Wp, Wc, Wz, Wx, bin_, W_ih, W_hh, bg, M2, WI, WJ, cst,
                W2blk, bm2, Wl1, Wl2, blat)
    in_specs = [
        pl.BlockSpec((1, N, N), lambda t: (t, 0, 0)),
        pl.BlockSpec((1, N, 4), lambda t: (t, 0, 0)),
    ] + [full(x) for x in operands[2:]]

    out = pl.pallas_call(
        _decoder_kernel,
        grid=(STEPS,),
        in_specs=in_specs,
        out_specs=pl.BlockSpec((1, N, 2), lambda t: (t, 0, 0)),
        out_shape=jax.ShapeDtypeStruct((STEPS, N, 2), F32),
        scratch_shapes=[
            pltpu.VMEM((N, 8), F32),
            pltpu.VMEM((N, 8), F32),
            pltpu.VMEM((N, 8), F32),
            pltpu.VMEM((N, 2), F32),
            pltpu.VMEM((N, 2), F32),
            pltpu.VMEM((N, 64), F32),
        ],
        compiler_params=pltpu.CompilerParams(
            dimension_semantics=("arbitrary",)),
    )(*operands)
    return out


# speedup vs baseline: 1.5073x; 1.4707x over previous
import jax
import jax.numpy as jnp
from jax.experimental import pallas as pl
from jax.experimental.pallas import tpu as pltpu

import numpy as np

N = 512
STEPS = 12
IB = 128
F32 = jnp.float32

_k1, _k2, _k3 = jax.random.split(jax.random.key(1), 3)
_H0 = np.asarray(jax.random.normal(_k1, (N, 8), dtype=F32))
_CC0 = np.asarray(jax.random.normal(_k2, (N, 8), dtype=F32))
_EPS = np.stack([np.asarray(jax.random.normal(jax.random.fold_in(_k3, i),
                                              (N, 4), dtype=F32))
                 for i in range(STEPS)])


def _dot(a, b):
    return jax.lax.dot_general(a, b, (((1,), (0,)), ((), ())),
                               preferred_element_type=F32)


def _decoder_kernel(nei_ref, eps_ref, c_ref, z_ref, p0_ref, lp_ref, h0_ref,
                    cc0_ref, Wp_ref, Wc_ref, Wz_ref, Wx_ref, bin_ref,
                    Wih_ref, Whh_ref, bg_ref, M2_ref, WI_ref, WJ_ref,
                    cst_ref, Wm2_ref, bm2_ref, Wl1_ref, Wl2_ref, blat_ref,
                    out_ref, h_s, cc_s, ctx_s, p_s, prev_s, A_s):
    t = pl.program_id(0)

    @pl.when(t == 0)
    def _init():
        h_s[...] = h0_ref[...]
        cc_s[...] = cc0_ref[...]
        ctx_s[...] = jnp.zeros((N, 8), F32)
        p_s[...] = p0_ref[...]
        prev_s[...] = lp_ref[...]

    h = h_s[...]
    cc = cc_s[...]
    ctx = ctx_s[...]
    p = p_s[...]
    prev = prev_s[...]

    emb = jax.nn.relu(_dot(prev, Wp_ref[...]) + _dot(c_ref[...], Wc_ref[...])
                      + _dot(z_ref[...], Wz_ref[...]) + _dot(ctx, Wx_ref[...])
                      + bin_ref[...])

    gates = _dot(emb, Wih_ref[...]) + _dot(h, Whh_ref[...]) + bg_ref[...]
    ig = jax.nn.sigmoid(gates[:, 0:8])
    fg = jax.nn.sigmoid(gates[:, 8:16])
    gg = jnp.tanh(gates[:, 16:24])
    og = jax.nn.sigmoid(gates[:, 24:32])
    cc = fg * cc + ig * gg
    h = og * jnp.tanh(cc)

    pm = _dot(p, M2_ref[...])
    A_s[...] = pm + _dot(h, WI_ref[...]) + cst_ref[...]
    B = _dot(h, WJ_ref[...]) - pm
    Bt = jnp.transpose(jnp.concatenate([B, B], axis=1))

    W2T = Wm2_ref[...]
    bm2 = bm2_ref[...]
    HN = N // 2

    def _block(b, carry):
        lo = b * IB
        Ab = jnp.concatenate([A_s[pl.ds(lo, IB), :],
                              A_s[pl.ds(HN + lo, IB), :]], axis=1)
        AbT = jnp.transpose(Ab)
        mk_lo = nei_ref[0, pl.ds(lo, IB), :]
        mk_hi = nei_ref[0, pl.ds(HN + lo, IB), :]
        x1big = jnp.concatenate(
            [jax.nn.relu(AbT[:, i2:i2 + 1] + Bt) for i2 in range(IB)],
            axis=1)
        x2big = _dot(W2T, x1big)
        pools = []
        for i2 in range(IB):
            x2t = x2big[:, i2 * N:(i2 + 1) * N]
            m16 = jnp.concatenate(
                [jnp.broadcast_to(mk_lo[i2][None, :], (8, N)),
                 jnp.broadcast_to(mk_hi[i2][None, :], (8, N))], axis=0)
            x2m = jnp.where(m16 > 0, x2t, -1e9)
            pools.append(jnp.max(x2m, axis=1))
        pool = jax.nn.relu(jnp.stack(pools, axis=1) + bm2)
        ctx_s[pl.ds(lo, IB), :] = jnp.transpose(pool[0:8, :])
        ctx_s[pl.ds(HN + lo, IB), :] = jnp.transpose(pool[8:16, :])
        return carry

    jax.lax.fori_loop(0, HN // IB, _block, 0)

    ctx = ctx_s[...]

    mean = h[:, 0:4]
    logvar = h[:, 4:8]
    lat = mean + eps_ref[0] * jnp.exp(0.5 * logvar)
    out = _dot(lat, Wl1_ref[...]) + _dot(ctx, Wl2_ref[...]) + blat_ref[...]

    out_ref[0] = out
    p_s[...] = p + out
    prev_s[...] = out
    h_s[...] = h
    cc_s[...] = cc


def kernel(last_position, c, z, obs_traj_pos, nei_index, nei_num_index,
           W_in, b_in, W_ih, W_hh, b_ih, b_hh, W_lat, b_lat, W_sp, b_sp,
           W_m1, b_m1, W_m2, b_m2):
    h0 = jnp.asarray(_H0)
    cc0 = jnp.asarray(_CC0)
    eps = jnp.asarray(_EPS)
    p0 = obs_traj_pos[-1]

    Wp = W_in[0:2]
    Wc = W_in[2:18]
    Wz = W_in[18:34]
    Wx = W_in[34:42]
    bin_ = b_in.reshape(1, 32)
    bg = (b_ih + b_hh).reshape(1, 32)
    M2 = W_sp @ W_m1[0:32]
    cst = (b_sp @ W_m1[0:32] + b_m1).reshape(1, 64)
    WJ = W_m1[32:40]
    WI = W_m1[40:48]
    W2blk = jnp.zeros((128, 16), F32)
    W2blk = W2blk.at[0:64, 0:8].set(W_m2).at[64:128, 8:16].set(W_m2)
    W2blk = W2blk.T
    bm2 = jnp.concatenate([b_m2, b_m2]).reshape(16, 1)
    Wl1 = W_lat[0:4]
    Wl2 = W_lat[4:12]
    blat = b_lat.reshape(1, 2)

    def full(x):
        nd = x.ndim
        return pl.BlockSpec(x.shape, lambda t, _nd=nd: (0,) * _nd)

    operands = (nei_index, eps, c, z, p0, last_position, h0, cc0,
                Wp, Wc, Wz, Wx, bin_, W_ih, W_hh, bg, M2, WI, WJ, cst,
                W2blk, bm2, Wl1, Wl2, blat)
    in_specs = [
        pl.BlockSpec((1, N, N), lambda t: (t, 0, 0)),
        pl.BlockSpec((1, N, 4), lambda t: (t, 0, 0)),
    ] + [full(x) for x in operands[2:]]

    out = pl.pallas_call(
        _decoder_kernel,
        grid=(STEPS,),
        in_specs=in_specs,
        out_specs=pl.BlockSpec((1, N, 2), lambda t: (t, 0, 0)),
        out_shape=jax.ShapeDtypeStruct((STEPS, N, 2), F32),
        scratch_shapes=[
            pltpu.VMEM((N, 8), F32),
            pltpu.VMEM((N, 8), F32),
            pltpu.VMEM((N, 8), F32),
            pltpu.VMEM((N, 2), F32),
            pltpu.VMEM((N, 2), F32),
            pltpu.VMEM((N, 64), F32),
        ],
        compiler_params=pltpu.CompilerParams(
            dimension_semantics=("arbitrary",)),
    )(*operands)
    return out


# speedup vs baseline: 1.7039x; 1.1305x over previous
import jax
import jax.numpy as jnp
from jax.experimental import pallas as pl
from jax.experimental.pallas import tpu as pltpu
import numpy as np

N = 512
STEPS = 12
IB = 128
F32 = jnp.float32

_k1, _k2, _k3 = jax.random.split(jax.random.key(1), 3)
_H0 = np.asarray(jax.random.normal(_k1, (N, 8), dtype=F32))
_CC0 = np.asarray(jax.random.normal(_k2, (N, 8), dtype=F32))
_EPS = np.stack([np.asarray(jax.random.normal(jax.random.fold_in(_k3, i),
                                              (N, 4), dtype=F32))
                 for i in range(STEPS)])


def _dot(a, b):
    return jax.lax.dot_general(a, b, (((1,), (0,)), ((), ())),
                               preferred_element_type=F32)


def _decoder_kernel(nei_ref, eps_ref, c_ref, z_ref, p0_ref, lp_ref, h0_ref,
                    cc0_ref, Win_ref, bin_ref, Wih_ref, Whh_ref, bih_ref,
                    bhh_ref, Wlat_ref, blat_ref, Wsp_ref, bsp_ref, Wm1_ref,
                    bm1_ref, Wm2_ref, bm2_ref,
                    out_ref, h_s, cc_s, ctx_s, p_s, prev_s, A_s):
    t = pl.program_id(0)

    @pl.when(t == 0)
    def _init():
        h_s[...] = h0_ref[...]
        cc_s[...] = cc0_ref[...]
        ctx_s[...] = jnp.zeros((N, 8), F32)
        p_s[...] = p0_ref[...]
        prev_s[...] = lp_ref[...]

    h = h_s[...]
    cc = cc_s[...]
    ctx = ctx_s[...]
    p = p_s[...]
    prev = prev_s[...]

    Win = Win_ref[...]
    Wm1 = Wm1_ref[...]
    M2 = _dot(Wsp_ref[...], Wm1[0:32])
    cst = _dot(bsp_ref[...], Wm1[0:32]) + bm1_ref[...]
    WJ = Wm1[32:40]
    WI = Wm1[40:48]
    Wm2T = jnp.transpose(Wm2_ref[...])
    zpad = jnp.zeros((8, 64), F32)
    W2T = jnp.concatenate(
        [jnp.concatenate([Wm2T, zpad], axis=1),
         jnp.concatenate([zpad, Wm2T], axis=1)], axis=0)
    bm2 = jnp.transpose(bm2_ref[...])
    bm2 = jnp.concatenate([bm2, bm2], axis=0)

    emb = jax.nn.relu(_dot(prev, Win[0:2]) + _dot(c_ref[...], Win[2:18])
                      + _dot(z_ref[...], Win[18:34]) + _dot(ctx, Win[34:42])
                      + bin_ref[...])

    gates = (_dot(emb, Wih_ref[...]) + _dot(h, Whh_ref[...])
             + bih_ref[...] + bhh_ref[...])
    ig = jax.nn.sigmoid(gates[:, 0:8])
    fg = jax.nn.sigmoid(gates[:, 8:16])
    gg = jnp.tanh(gates[:, 16:24])
    og = jax.nn.sigmoid(gates[:, 24:32])
    cc = fg * cc + ig * gg
    h = og * jnp.tanh(cc)

    pm = _dot(p, M2)
    A_s[...] = pm + _dot(h, WI) + cst
    B = _dot(h, WJ) - pm
    Bt = jnp.transpose(jnp.concatenate([B, B], axis=1))

    HN = N // 2

    def _block(b, carry):
        lo = b * IB
        Ab = jnp.concatenate([A_s[pl.ds(lo, IB), :],
                              A_s[pl.ds(HN + lo, IB), :]], axis=1)
        AbT = jnp.transpose(Ab)
        mk_lo = nei_ref[0, pl.ds(lo, IB), :]
        mk_hi = nei_ref[0, pl.ds(HN + lo, IB), :]
        x1big = jnp.concatenate(
            [jax.nn.relu(AbT[:, i2:i2 + 1] + Bt) for i2 in range(IB)],
            axis=1)
        x2big = _dot(W2T, x1big)
        pools = []
        for i2 in range(IB):
            x2t = x2big[:, i2 * N:(i2 + 1) * N]
            m16 = jnp.concatenate(
                [jnp.broadcast_to(mk_lo[i2][None, :], (8, N)),
                 jnp.broadcast_to(mk_hi[i2][None, :], (8, N))], axis=0)
            x2m = jnp.where(m16 > 0, x2t, -1e9)
            pools.append(jnp.max(x2m, axis=1))
        pool = jax.nn.relu(jnp.stack(pools, axis=1) + bm2)
        ctx_s[pl.ds(lo, IB), :] = jnp.transpose(pool[0:8, :])
        ctx_s[pl.ds(HN + lo, IB), :] = jnp.transpose(pool[8:16, :])
        return carry

    jax.lax.fori_loop(0, HN // IB, _block, 0)

    ctx = ctx_s[...]

    Wlat = Wlat_ref[...]
    mean = h[:, 0:4]
    logvar = h[:, 4:8]
    lat = mean + eps_ref[0] * jnp.exp(0.5 * logvar)
    out = _dot(lat, Wlat[0:4]) + _dot(ctx, Wlat[4:12]) + blat_ref[...]

    out_ref[0] = out
    p_s[...] = p + out
    prev_s[...] = out
    h_s[...] = h
    cc_s[...] = cc


def kernel(last_position, c, z, obs_traj_pos, nei_index, nei_num_index,
           W_in, b_in, W_ih, W_hh, b_ih, b_hh, W_lat, b_lat, W_sp, b_sp,
           W_m1, b_m1, W_m2, b_m2):
    h0 = jnp.asarray(_H0)
    cc0 = jnp.asarray(_CC0)
    eps = jnp.asarray(_EPS)
    p0 = obs_traj_pos[-1]

    operands = (nei_index, eps, c, z, p0, last_position, h0, cc0,
                W_in, b_in.reshape(1, 32), W_ih, W_hh,
                b_ih.reshape(1, 32), b_hh.reshape(1, 32),
                W_lat, b_lat.reshape(1, 2), W_sp, b_sp.reshape(1, 32),
                W_m1, b_m1.reshape(1, 64), W_m2, b_m2.reshape(1, 8))

    def full(x):
        nd = x.ndim
        return pl.BlockSpec(x.shape, lambda t, _nd=nd: (0,) * _nd)

    in_specs = [
        pl.BlockSpec((1, N, N), lambda t: (t, 0, 0)),
        pl.BlockSpec((1, N, 4), lambda t: (t, 0, 0)),
    ] + [full(x) for x in operands[2:]]

    out = pl.pallas_call(
        _decoder_kernel,
        grid=(STEPS,),
        in_specs=in_specs,
        out_specs=pl.BlockSpec((1, N, 2), lambda t: (t, 0, 0)),
        out_shape=jax.ShapeDtypeStruct((STEPS, N, 2), F32),
        scratch_shapes=[
            pltpu.VMEM((N, 8), F32),
            pltpu.VMEM((N, 8), F32),
            pltpu.VMEM((N, 8), F32),
            pltpu.VMEM((N, 2), F32),
            pltpu.VMEM((N, 2), F32),
            pltpu.VMEM((N, 64), F32),
        ],
        compiler_params=pltpu.CompilerParams(
            dimension_semantics=("arbitrary",)),
    )(*operands)
    return out
